# baseline (speedup 1.0000x reference)
import functools

import jax
import jax.numpy as jnp
from jax import lax
from jax.experimental import pallas as pl
from jax.experimental.pallas import tpu as pltpu
from jax.experimental.pallas import tpu_sc as plsc

_GH, _GW = 32, 32
_LATENT = 256
_ALPHA = 1.0
_TIME_DECAY = 0.9

_F32 = jnp.float32


def _dot(a, b, dims, precision=lax.Precision.HIGHEST):
    return lax.dot_general(
        a, b, dimension_numbers=(dims, ((), ())),
        preferred_element_type=_F32, precision=precision)


def _safe_dist(sq):
    return jnp.sqrt(jnp.maximum(sq, 0.0))


def _pass1_body(T, K, nblocks, z_ref, nodes_ref, tw_ref, rsq_ref, nsq_ref,
                q_ref, bmu_ref, s_ref, misc_ref):
    i = pl.program_id(0)
    zb = z_ref[...]
    nod = nodes_ref[...]

    @pl.when(i == 0)
    def _init():
        s_ref[...] = jnp.zeros_like(s_ref)
        misc_ref[...] = jnp.zeros_like(misc_ref)
        n2c = jnp.sum(nod * nod, axis=1, keepdims=True)
        dmm = _dot(nod, nod, ((1,), (1,)), lax.Precision.DEFAULT)
        ndist = _safe_dist(n2c + nsq_ref[0:1, :] - 2.0 * dmm)
        misc_ref[0:1, 2:3] = -jnp.sum(ndist, axis=(0, 1),
                                      keepdims=True) / _F32(K * K)

    tw = tw_ref[...]
    wz = zb * tw

    row_sq = rsq_ref[...]
    node_sq = nsq_ref[0:1, :]
    mm = _dot(wz, nod, ((1,), (1,)), precision=lax.Precision.DEFAULT)
    dist = _safe_dist(row_sq + node_sq - 2.0 * mm)

    qu = 1.0 / (1.0 + dist)
    qn = qu * (1.0 / jnp.sum(qu, axis=1, keepdims=True))
    q_ref[...] = qn
    s_ref[0:1, :] += jnp.sum(qn * qn, axis=0, keepdims=True)

    minv = jnp.min(dist, axis=1, keepdims=True)
    jcol = lax.broadcasted_iota(jnp.int32, (T, K), 1)
    idx = jnp.min(jnp.where(dist == minv, jcol, K), axis=1, keepdims=True)
    bmu_ref[...] = idx

    dz = zb[1:, :] - zb[:-1, :]
    misc_ref[0:1, 0:1] += jnp.sum(dz * dz, axis=(0, 1), keepdims=True)

    r = idx // _GW
    c = idx % _GW
    md = jnp.abs(r[1:] - r[:-1]) + jnp.abs(c[1:] - c[:-1])
    misc_ref[0:1, 1:2] += jnp.sum(md, axis=(0, 1), keepdims=True).astype(_F32)


def _pass2_body(N, nblocks, q_ref, s_ref, kl_ref):
    i = pl.program_id(0)

    @pl.when(i == 0)
    def _init():
        kl_ref[...] = jnp.zeros_like(kl_ref)

    qb = q_ref[...]
    s = s_ref[0:1, :]
    a = qb * qb / s
    r = jnp.sum(a, axis=1, keepdims=True)
    t1 = jnp.sum(a * (jnp.log(qb) - jnp.log(s)), axis=1, keepdims=True)
    part = t1 / r - jnp.log(r)
    kl_ref[0:1, 0:1] += jnp.sum(part, axis=(0, 1), keepdims=True)

    @pl.when(i == nblocks - 1)
    def _fin():
        kl_ref[0:1, 0:1] = kl_ref[0:1, 0:1] / _F32(N)


def _som_sc(z_flat, nodes_flat, bmu_flat):
    N, D = z_flat.shape
    info = plsc.get_sparse_core_info()
    nc, ns = info.num_cores, info.num_subcores
    nw = nc * ns
    b_per_w = N // nw
    ch = 64
    nch = b_per_w // ch
    mesh = plsc.VectorSubcoreMesh(core_axis_name="c", subcore_axis_name="s")

    @functools.partial(
        pl.kernel, mesh=mesh,
        out_type=jax.ShapeDtypeStruct((N, D), _F32),
        scratch_types=[
            pltpu.VMEM((b_per_w,), jnp.int32),
            pltpu.VMEM((ch, D), _F32),
            pltpu.VMEM((ch, D), _F32),
            pltpu.SemaphoreType.DMA,
        ],
    )
    def k(z_hbm, nodes_hbm, idx_hbm, som_hbm, idx_v, rows_v, z_v, sem):
        wid = lax.axis_index("s") * nc + lax.axis_index("c")
        base = wid * b_per_w
        pltpu.sync_copy(idx_hbm.at[pl.ds(base, b_per_w)], idx_v)
        for c in range(nch):
            pltpu.async_copy(
                nodes_hbm.at[idx_v.at[pl.ds(c * ch, ch)]], rows_v, sem
            ).wait()
            pltpu.sync_copy(z_hbm.at[pl.ds(base + c * ch, ch)], z_v)

            @plsc.parallel_loop(0, ch, 1, unroll=4)
            def _blend(r):
                for j in range(D // 16):
                    zv = z_v[r, pl.ds(j * 16, 16)]
                    rv = rows_v[r, pl.ds(j * 16, 16)]
                    rows_v[r, pl.ds(j * 16, 16)] = zv + 0.1 * (rv - zv)
            pltpu.sync_copy(rows_v, som_hbm.at[pl.ds(base + c * ch, ch)])

    return k(z_flat, nodes_flat, bmu_flat)


def kernel(z, nodes):
    B, T, D = z.shape
    K = nodes.shape[0] * nodes.shape[1]
    N = B * T
    z_flat = z.reshape(N, D)
    nodes_flat = nodes.reshape(K, D)
    tw = (_TIME_DECAY ** (T - jnp.arange(T, dtype=_F32) - 1.0)).reshape(T, 1)
    wz_flat = (z * tw.reshape(1, T, 1)).reshape(N, D)
    row_sq = jnp.sum(wz_flat * wz_flat, axis=1).reshape(N, 1)
    node_sq = jnp.sum(nodes_flat * nodes_flat, axis=1).reshape(1, K)

    grid = B
    q, bmu, s_out, misc = pl.pallas_call(
        functools.partial(_pass1_body, T, K, grid),
        grid=(grid,),
        in_specs=[
            pl.BlockSpec((T, D), lambda i: (i, 0)),
            pl.BlockSpec((K, D), lambda i: (0, 0)),
            pl.BlockSpec((T, 1), lambda i: (0, 0)),
            pl.BlockSpec((T, 1), lambda i: (i, 0)),
            pl.BlockSpec((1, K), lambda i: (0, 0)),
        ],
        out_specs=[
            pl.BlockSpec((T, K), lambda i: (i, 0)),
            pl.BlockSpec((T, 1), lambda i: (i, 0)),
            pl.BlockSpec((8, K), lambda i: (0, 0)),
            pl.BlockSpec((8, 128), lambda i: (0, 0)),
        ],
        out_shape=[
            jax.ShapeDtypeStruct((N, K), _F32),
            jax.ShapeDtypeStruct((N, 1), jnp.int32),
            jax.ShapeDtypeStruct((8, K), _F32),
            jax.ShapeDtypeStruct((8, 128), _F32),
        ],
        compiler_params=pltpu.CompilerParams(
            dimension_semantics=("arbitrary",)),
    )(z_flat, nodes_flat, tw, row_sq, node_sq)

    som = _som_sc(z_flat, nodes_flat, bmu.reshape(N))

    kl_out = pl.pallas_call(
        functools.partial(_pass2_body, N, grid),
        grid=(grid,),
        in_specs=[
            pl.BlockSpec((T, K), lambda i: (i, 0)),
            pl.BlockSpec((8, K), lambda i: (0, 0)),
        ],
        out_specs=pl.BlockSpec((8, 128), lambda i: (0, 0)),
        out_shape=jax.ShapeDtypeStruct((8, 128), _F32),
        compiler_params=pltpu.CompilerParams(
            dimension_semantics=("arbitrary",)),
    )(q, s_out)

    kl_loss = kl_out[0, 0]
    diversity_loss = misc[0, 2]
    time_smooth_loss = misc[0, 0] * (_TIME_DECAY / (B * (T - 1) * D))
    neighbor_loss = misc[0, 1] / ((T - 1) * B)
    total_loss = (kl_loss + 0.5 * diversity_loss
                  + 0.3 * time_smooth_loss + 0.2 * neighbor_loss)
    return (som.reshape(B, T, D), total_loss, kl_loss, diversity_loss,
            time_smooth_loss, neighbor_loss, q, bmu.reshape(B, T))


# speedup vs baseline: 1.0042x; 1.0042x over previous
import functools

import jax
import jax.numpy as jnp
from jax import lax
from jax.experimental import pallas as pl
from jax.experimental.pallas import tpu as pltpu
from jax.experimental.pallas import tpu_sc as plsc

_GH, _GW = 32, 32
_LATENT = 256
_ALPHA = 1.0
_TIME_DECAY = 0.9

_F32 = jnp.float32


def _dot(a, b, dims, precision=lax.Precision.HIGHEST):
    return lax.dot_general(
        a, b, dimension_numbers=(dims, ((), ())),
        preferred_element_type=_F32, precision=precision)


def _safe_dist(sq):
    return jnp.sqrt(jnp.maximum(sq, 0.0))


def _pass1_body(T, K, nblocks, z_ref, nodes_ref, tw_ref, rsq_ref, nsq_ref,
                q_ref, bmu_ref, s_ref, misc_ref):
    i = pl.program_id(0)
    zb = z_ref[...]
    nod = nodes_ref[...]

    @pl.when(i == 0)
    def _init():
        s_ref[...] = jnp.zeros_like(s_ref)
        misc_ref[...] = jnp.zeros_like(misc_ref)
        n2c = jnp.sum(nod * nod, axis=1, keepdims=True)
        dmm = _dot(nod, nod, ((1,), (1,)), lax.Precision.DEFAULT)
        ndist = _safe_dist(n2c + nsq_ref[0:1, :] - 2.0 * dmm)
        misc_ref[0:1, 2:3] = -jnp.sum(ndist, axis=(0, 1),
                                      keepdims=True) / _F32(K * K)

    tw = tw_ref[...]
    wz = zb * tw

    row_sq = rsq_ref[...]
    node_sq = nsq_ref[0:1, :]
    mm = _dot(wz, nod, ((1,), (1,)), precision=lax.Precision.DEFAULT)
    dist = _safe_dist(row_sq + node_sq - 2.0 * mm)

    qu = 1.0 / (1.0 + dist)
    qn = qu * (1.0 / jnp.sum(qu, axis=1, keepdims=True))
    q_ref[...] = qn
    s_ref[0:1, :] += jnp.sum(qn * qn, axis=0, keepdims=True)

    minv = jnp.min(dist, axis=1, keepdims=True)
    jcol = lax.broadcasted_iota(jnp.int32, (T, K), 1)
    idx = jnp.min(jnp.where(dist == minv, jcol, K), axis=1, keepdims=True)
    bmu_ref[...] = idx

    dz = zb[1:, :] - zb[:-1, :]
    misc_ref[0:1, 0:1] += jnp.sum(dz * dz, axis=(0, 1), keepdims=True)

    r = idx // _GW
    c = idx % _GW
    md = jnp.abs(r[1:] - r[:-1]) + jnp.abs(c[1:] - c[:-1])
    misc_ref[0:1, 1:2] += jnp.sum(md, axis=(0, 1), keepdims=True).astype(_F32)


def _pass2_body(N, nblocks, q_ref, s_ref, kl_ref):
    i = pl.program_id(0)

    @pl.when(i == 0)
    def _init():
        kl_ref[...] = jnp.zeros_like(kl_ref)

    qb = q_ref[...]
    s = s_ref[0:1, :]
    a = qb * qb / s
    r = jnp.sum(a, axis=1, keepdims=True)
    t1 = jnp.sum(a * (jnp.log(qb) - jnp.log(s)), axis=1, keepdims=True)
    part = t1 / r - jnp.log(r)
    kl_ref[0:1, 0:1] += jnp.sum(part, axis=(0, 1), keepdims=True)

    @pl.when(i == nblocks - 1)
    def _fin():
        kl_ref[0:1, 0:1] = kl_ref[0:1, 0:1] / _F32(N)


def _som_sc(z_flat, nodes_flat, bmu_flat):
    N, D = z_flat.shape
    info = plsc.get_sparse_core_info()
    nc, ns = info.num_cores, info.num_subcores
    nw = nc * ns
    b_per_w = N // nw
    ch = 128
    nch = b_per_w // ch
    mesh = plsc.VectorSubcoreMesh(core_axis_name="c", subcore_axis_name="s")

    @functools.partial(
        pl.kernel, mesh=mesh,
        out_type=jax.ShapeDtypeStruct((N, D), _F32),
        scratch_types=[
            pltpu.VMEM((b_per_w,), jnp.int32),
            pltpu.VMEM((ch, D), _F32),
            pltpu.VMEM((ch, D), _F32),
            pltpu.SemaphoreType.DMA,
        ],
    )
    def k(z_hbm, nodes_hbm, idx_hbm, som_hbm, idx_v, rows_v, z_v, sem):
        wid = lax.axis_index("s") * nc + lax.axis_index("c")
        base = wid * b_per_w
        pltpu.sync_copy(idx_hbm.at[pl.ds(base, b_per_w)], idx_v)
        for c in range(nch):
            pltpu.async_copy(
                nodes_hbm.at[idx_v.at[pl.ds(c * ch, ch)]], rows_v, sem
            ).wait()
            pltpu.sync_copy(z_hbm.at[pl.ds(base + c * ch, ch)], z_v)

            @plsc.parallel_loop(0, ch, 1, unroll=4)
            def _blend(r):
                for j in range(D // 16):
                    zv = z_v[r, pl.ds(j * 16, 16)]
                    rv = rows_v[r, pl.ds(j * 16, 16)]
                    rows_v[r, pl.ds(j * 16, 16)] = zv + 0.1 * (rv - zv)
            pltpu.sync_copy(rows_v, som_hbm.at[pl.ds(base + c * ch, ch)])

    return k(z_flat, nodes_flat, bmu_flat)


def kernel(z, nodes):
    B, T, D = z.shape
    K = nodes.shape[0] * nodes.shape[1]
    N = B * T
    z_flat = z.reshape(N, D)
    nodes_flat = nodes.reshape(K, D)
    tw = (_TIME_DECAY ** (T - jnp.arange(T, dtype=_F32) - 1.0)).reshape(T, 1)
    wz_flat = (z * tw.reshape(1, T, 1)).reshape(N, D)
    row_sq = jnp.sum(wz_flat * wz_flat, axis=1).reshape(N, 1)
    node_sq = jnp.sum(nodes_flat * nodes_flat, axis=1).reshape(1, K)

    grid = B
    q, bmu, s_out, misc = pl.pallas_call(
        functools.partial(_pass1_body, T, K, grid),
        grid=(grid,),
        in_specs=[
            pl.BlockSpec((T, D), lambda i: (i, 0)),
            pl.BlockSpec((K, D), lambda i: (0, 0)),
            pl.BlockSpec((T, 1), lambda i: (0, 0)),
            pl.BlockSpec((T, 1), lambda i: (i, 0)),
            pl.BlockSpec((1, K), lambda i: (0, 0)),
        ],
        out_specs=[
            pl.BlockSpec((T, K), lambda i: (i, 0)),
            pl.BlockSpec((T, 1), lambda i: (i, 0)),
            pl.BlockSpec((8, K), lambda i: (0, 0)),
            pl.BlockSpec((8, 128), lambda i: (0, 0)),
        ],
        out_shape=[
            jax.ShapeDtypeStruct((N, K), _F32),
            jax.ShapeDtypeStruct((N, 1), jnp.int32),
            jax.ShapeDtypeStruct((8, K), _F32),
            jax.ShapeDtypeStruct((8, 128), _F32),
        ],
        compiler_params=pltpu.CompilerParams(
            dimension_semantics=("arbitrary",)),
    )(z_flat, nodes_flat, tw, row_sq, node_sq)

    som = _som_sc(z_flat, nodes_flat, bmu.reshape(N))

    kl_out = pl.pallas_call(
        functools.partial(_pass2_body, N, grid),
        grid=(grid,),
        in_specs=[
            pl.BlockSpec((T, K), lambda i: (i, 0)),
            pl.BlockSpec((8, K), lambda i: (0, 0)),
        ],
        out_specs=pl.BlockSpec((8, 128), lambda i: (0, 0)),
        out_shape=jax.ShapeDtypeStruct((8, 128), _F32),
        compiler_params=pltpu.CompilerParams(
            dimension_semantics=("arbitrary",)),
    )(q, s_out)

    kl_loss = kl_out[0, 0]
    diversity_loss = misc[0, 2]
    time_smooth_loss = misc[0, 0] * (_TIME_DECAY / (B * (T - 1) * D))
    neighbor_loss = misc[0, 1] / ((T - 1) * B)
    total_loss = (kl_loss + 0.5 * diversity_loss
                  + 0.3 * time_smooth_loss + 0.2 * neighbor_loss)
    return (som.reshape(B, T, D), total_loss, kl_loss, diversity_loss,
            time_smooth_loss, neighbor_loss, q, bmu.reshape(B, T))


# speedup vs baseline: 3.9594x; 3.9430x over previous
import functools

import jax
import jax.numpy as jnp
from jax import lax
from jax.experimental import pallas as pl
from jax.experimental.pallas import tpu as pltpu

_GH, _GW = 32, 32
_LATENT = 256
_ALPHA = 1.0
_TIME_DECAY = 0.9

_F32 = jnp.float32


def _dot(a, b, dims, precision=lax.Precision.HIGHEST):
    return lax.dot_general(
        a, b, dimension_numbers=(dims, ((), ())),
        preferred_element_type=_F32, precision=precision)


def _safe_dist(sq):
    return jnp.sqrt(jnp.maximum(sq, 0.0))


def _pass1_body(T, K, nblocks, z_ref, nodes_ref, nhi_ref, nlo_ref, nlo2_ref,
                tw_ref, nsq_ref, q_ref, som_ref, bmu_ref, s_ref,
                misc_ref):
    i = pl.program_id(0)
    zb = z_ref[...]
    nod = nodes_ref[...]

    @pl.when(i == 0)
    def _init():
        s_ref[...] = jnp.zeros_like(s_ref)
        misc_ref[...] = jnp.zeros_like(misc_ref)
        n2c = jnp.sum(nod * nod, axis=1, keepdims=True)
        dmm = _dot(nod, nod, ((1,), (1,)), lax.Precision.DEFAULT)
        ndist = _safe_dist(n2c + nsq_ref[0:1, :] - 2.0 * dmm)
        misc_ref[0:1, 2:3] = -jnp.sum(ndist, axis=(0, 1),
                                      keepdims=True) / _F32(K * K)

    tw = tw_ref[...]
    wz = zb * tw

    row_sq = jnp.sum(wz * wz, axis=1, keepdims=True)
    node_sq = nsq_ref[0:1, :]
    mm = _dot(wz, nod, ((1,), (1,)), precision=lax.Precision.DEFAULT)
    dist = _safe_dist(row_sq + node_sq - 2.0 * mm)

    qu = 1.0 / (1.0 + dist)
    qn = qu * (1.0 / jnp.sum(qu, axis=1, keepdims=True))
    q_ref[...] = qn
    s_ref[0:1, :] += jnp.sum(qn * qn, axis=0, keepdims=True)

    minv = jnp.min(dist, axis=1, keepdims=True)
    jcol = lax.broadcasted_iota(jnp.int32, (T, K), 1)
    idx = jnp.min(jnp.where(dist == minv, jcol, K), axis=1, keepdims=True)
    bmu_ref[...] = idx

    oh = (jcol == idx).astype(jnp.bfloat16)
    bnodes = (_dot(oh, nhi_ref[...], ((1,), (0,)), lax.Precision.DEFAULT)
              + _dot(oh, nlo_ref[...], ((1,), (0,)), lax.Precision.DEFAULT)
              + _dot(oh, nlo2_ref[...], ((1,), (0,)), lax.Precision.DEFAULT))
    som_ref[...] = zb + 0.1 * (bnodes - zb)

    dz = zb[1:, :] - zb[:-1, :]
    misc_ref[0:1, 0:1] += jnp.sum(dz * dz, axis=(0, 1), keepdims=True)

    r = idx // _GW
    c = idx % _GW
    md = jnp.abs(r[1:] - r[:-1]) + jnp.abs(c[1:] - c[:-1])
    misc_ref[0:1, 1:2] += jnp.sum(md, axis=(0, 1), keepdims=True).astype(_F32)


def _pass2_body(N, nblocks, q_ref, s_ref, kl_ref):
    i = pl.program_id(0)

    @pl.when(i == 0)
    def _init():
        kl_ref[...] = jnp.zeros_like(kl_ref)

    qb = q_ref[...]
    s = s_ref[0:1, :]
    a = qb * qb / s
    r = jnp.sum(a, axis=1, keepdims=True)
    t1 = jnp.sum(a * (jnp.log(qb) - jnp.log(s)), axis=1, keepdims=True)
    part = t1 / r - jnp.log(r)
    kl_ref[0:1, 0:1] += jnp.sum(part, axis=(0, 1), keepdims=True)

    @pl.when(i == nblocks - 1)
    def _fin():
        kl_ref[0:1, 0:1] = kl_ref[0:1, 0:1] / _F32(N)


def kernel(z, nodes):
    B, T, D = z.shape
    K = nodes.shape[0] * nodes.shape[1]
    N = B * T
    z_flat = z.reshape(N, D)
    nodes_flat = nodes.reshape(K, D)
    tw = (_TIME_DECAY ** (T - jnp.arange(T, dtype=_F32) - 1.0)).reshape(T, 1)
    node_sq = jnp.sum(nodes_flat * nodes_flat, axis=1).reshape(1, K)
    n_hi = nodes_flat.astype(jnp.bfloat16)
    rem = nodes_flat - n_hi.astype(_F32)
    n_lo = rem.astype(jnp.bfloat16)
    n_lo2 = (rem - n_lo.astype(_F32)).astype(jnp.bfloat16)

    grid = B
    q, som, bmu, s_out, misc = pl.pallas_call(
        functools.partial(_pass1_body, T, K, grid),
        grid=(grid,),
        in_specs=[
            pl.BlockSpec((T, D), lambda i: (i, 0)),
            pl.BlockSpec((K, D), lambda i: (0, 0)),
            pl.BlockSpec((K, D), lambda i: (0, 0)),
            pl.BlockSpec((K, D), lambda i: (0, 0)),
            pl.BlockSpec((K, D), lambda i: (0, 0)),
            pl.BlockSpec((T, 1), lambda i: (0, 0)),
            pl.BlockSpec((1, K), lambda i: (0, 0)),
        ],
        out_specs=[
            pl.BlockSpec((T, K), lambda i: (i, 0)),
            pl.BlockSpec((T, D), lambda i: (i, 0)),
            pl.BlockSpec((T, 1), lambda i: (i, 0)),
            pl.BlockSpec((8, K), lambda i: (0, 0)),
            pl.BlockSpec((8, 128), lambda i: (0, 0)),
        ],
        out_shape=[
            jax.ShapeDtypeStruct((N, K), _F32),
            jax.ShapeDtypeStruct((N, D), _F32),
            jax.ShapeDtypeStruct((N, 1), jnp.int32),
            jax.ShapeDtypeStruct((8, K), _F32),
            jax.ShapeDtypeStruct((8, 128), _F32),
        ],
        compiler_params=pltpu.CompilerParams(
            dimension_semantics=("arbitrary",)),
    )(z_flat, nodes_flat, n_hi, n_lo, n_lo2, tw, node_sq)

    kl_out = pl.pallas_call(
        functools.partial(_pass2_body, N, grid),
        grid=(grid,),
        in_specs=[
            pl.BlockSpec((T, K), lambda i: (i, 0)),
            pl.BlockSpec((8, K), lambda i: (0, 0)),
        ],
        out_specs=pl.BlockSpec((8, 128), lambda i: (0, 0)),
        out_shape=jax.ShapeDtypeStruct((8, 128), _F32),
        compiler_params=pltpu.CompilerParams(
            dimension_semantics=("arbitrary",)),
    )(q, s_out)

    kl_loss = kl_out[0, 0]
    diversity_loss = misc[0, 2]
    time_smooth_loss = misc[0, 0] * (_TIME_DECAY / (B * (T - 1) * D))
    neighbor_loss = misc[0, 1] / ((T - 1) * B)
    total_loss = (kl_loss + 0.5 * diversity_loss
                  + 0.3 * time_smooth_loss + 0.2 * neighbor_loss)
    return (som.reshape(B, T, D), total_loss, kl_loss, diversity_loss,
            time_smooth_loss, neighbor_loss, q, bmu.reshape(B, T))
